# baseline (speedup 1.0000x reference)
import functools

import jax
import jax.numpy as jnp
from jax import lax
from jax.experimental import pallas as pl
from jax.experimental.pallas import tpu as pltpu
from jax.experimental.pallas import tpu_sc as plsc

D = 128
K = 10
NC = 2
NS = 16
NW = NC * NS
C = 16
BB = 8192
NBUF = 4


def _sc_gather_mean(idx2, features, Bp):
    mesh = plsc.VectorSubcoreMesh(core_axis_name="c", subcore_axis_name="s")

    @functools.partial(
        pl.kernel,
        out_type=(
            jax.ShapeDtypeStruct((Bp, D), jnp.float32),
            jax.ShapeDtypeStruct((Bp, D), jnp.float32),
        ),
        mesh=mesh,
        compiler_params=pltpu.CompilerParams(
            disable_bounds_checks=True, disable_semaphore_checks=True),
        scratch_types=(
            [pltpu.VMEM(((K + 1) * C,), jnp.int32) for _ in range(NBUF)]
            + [pltpu.VMEM((NBUF, (K + 1) * C, D), jnp.float32),
               pltpu.VMEM((NBUF, C, D), jnp.float32)]
            + [pltpu.SemaphoreType.DMA for _ in range(2 * NBUF + 1)]
        ),
    )
    def run(idx_hbm, feat_hbm, self_hbm, mean_hbm, *scr):
        idx_vs = scr[:NBUF]
        rows_v, mean_v = scr[NBUF], scr[NBUF + 1]
        sems_g = scr[NBUF + 2:2 * NBUF + 2]
        sems_i = scr[2 * NBUF + 2:3 * NBUF + 2]
        sem_wb = scr[3 * NBUF + 2]
        wid = lax.axis_index("s") * NC + lax.axis_index("c")
        count = (Bp // C) // NW
        start = wid * count

        def fire_idx(ci, b):
            pltpu.async_copy(idx_hbm.at[start + ci], idx_vs[b], sems_i[b])

        def wait_idx(b):
            pltpu.make_async_copy(idx_hbm.at[start], idx_vs[b],
                                  sems_i[b]).wait()

        def fire_gather(b):
            pltpu.async_copy(feat_hbm.at[idx_vs[b]], rows_v.at[b], sems_g[b])

        def wait_gather(b):
            pltpu.make_async_copy(feat_hbm.at[idx_vs[b]], rows_v.at[b],
                                  sems_g[b]).wait()

        def fire_wb(ci, b):
            base = (start + ci) * C
            pltpu.async_copy(rows_v.at[b].at[pl.ds(0, C)],
                             self_hbm.at[pl.ds(base, C)], sem_wb)
            pltpu.async_copy(mean_v.at[b], mean_hbm.at[pl.ds(base, C)], sem_wb)

        def drain_wb(b):
            pltpu.make_async_copy(rows_v.at[b].at[pl.ds(0, C)],
                                  self_hbm.at[pl.ds(0, C)], sem_wb).wait()
            pltpu.make_async_copy(mean_v.at[b],
                                  mean_hbm.at[pl.ds(0, C)], sem_wb).wait()

        def compute(b):
            def row_body(r, rc):
                for g in range(D // 16):
                    s = pl.ds(g * 16, 16)
                    acc = rows_v[b, C + r, s]
                    for j in range(2, K + 1):
                        acc = acc + rows_v[b, j * C + r, s]
                    mean_v[b, r, s] = acc * jnp.float32(1.0 / K)
                return rc

            lax.fori_loop(0, C, row_body, 0)

        with jax.named_scope("sc_prolog"):
            pltpu.sync_copy(idx_hbm.at[start], idx_vs[0])
            for m in range(1, NBUF):
                fire_idx(m, m)
            fire_gather(0)
            for m in range(1, NBUF - 1):
                wait_idx(m)
                fire_gather(m)

        def group_body(i2, carry):
            for b in range(NBUF):
                ci = NBUF * i2 + b
                pb = (b - 1) % NBUF
                wait_gather(b)
                pl.when(ci + NBUF < count)(lambda: fire_idx(ci + NBUF, b))
                pl.when(ci >= 1)(lambda: drain_wb(pb))
                pl.when(ci + NBUF - 1 < count)(lambda: wait_idx(pb))
                pl.when(ci + NBUF - 1 < count)(lambda: fire_gather(pb))
                compute(b)
                fire_wb(ci, b)
            return carry

        with jax.named_scope("sc_loop"):
            lax.fori_loop(0, count // NBUF, group_body, 0)
        with jax.named_scope("sc_epilog"):
            drain_wb(NBUF - 1)

    return run(idx2, features)


def _mm_body(xs_ref, xm_ref, w1_ref, w2_ref, o_ref):
    dn = (((1,), (1,)), ((), ()))
    a = lax.dot_general(w1_ref[...], xs_ref[...], dn,
                        preferred_element_type=jnp.float32)
    b = lax.dot_general(w2_ref[...], xm_ref[...], dn,
                        preferred_element_type=jnp.float32)
    o_ref[...] = jnp.maximum(a + b, 0.0)


def _mm(xs, xm, w1, w2, B):
    nb = (B + BB - 1) // BB
    return pl.pallas_call(
        _mm_body,
        grid=(nb,),
        in_specs=[
            pl.BlockSpec((BB, D), lambda i: (i, 0)),
            pl.BlockSpec((BB, D), lambda i: (i, 0)),
            pl.BlockSpec((D, D), lambda i: (0, 0)),
            pl.BlockSpec((D, D), lambda i: (0, 0)),
        ],
        out_specs=pl.BlockSpec((D, BB), lambda i: (0, i)),
        out_shape=jax.ShapeDtypeStruct((D, B), jnp.float32),
    )(xs, xm, w1, w2)


def kernel(nodes, neigh_idx, features, W):
    B = nodes.shape[0]
    step = NW * NBUF * C
    Bp = ((B + step - 1) // step) * step
    nchunks_total = Bp // C

    idxT = jnp.concatenate(
        [nodes[None, :].astype(jnp.int32), neigh_idx.T.astype(jnp.int32)], axis=0)
    pad = (jnp.arange((K + 1) * (Bp - B), dtype=jnp.int32)
           % features.shape[0]).reshape(K + 1, Bp - B)
    idxT = jnp.concatenate([idxT, pad], axis=1)
    idx2 = (idxT.reshape(K + 1, nchunks_total, C)
            .transpose(1, 0, 2)
            .reshape(nchunks_total, (K + 1) * C))

    self_f, mean_f = _sc_gather_mean(idx2, features, Bp)
    return _mm(self_f, mean_f, W[:, :D], W[:, D:], B)


# speedup vs baseline: 1.0140x; 1.0140x over previous
import functools

import jax
import jax.numpy as jnp
from jax import lax
from jax.experimental import pallas as pl
from jax.experimental.pallas import tpu as pltpu
from jax.experimental.pallas import tpu_sc as plsc

D = 128
K = 10
NC = 2
NS = 16
NW = NC * NS
C = 24
BB = 8192
NBUF = 3


def _sc_gather_mean(idx2, features, Bp):
    mesh = plsc.VectorSubcoreMesh(core_axis_name="c", subcore_axis_name="s")

    @functools.partial(
        pl.kernel,
        out_type=(
            jax.ShapeDtypeStruct((Bp, D), jnp.float32),
            jax.ShapeDtypeStruct((Bp, D), jnp.float32),
        ),
        mesh=mesh,
        compiler_params=pltpu.CompilerParams(
            disable_bounds_checks=True, disable_semaphore_checks=True),
        scratch_types=(
            [pltpu.VMEM(((K + 1) * C,), jnp.int32) for _ in range(NBUF)]
            + [pltpu.VMEM((NBUF, (K + 1) * C, D), jnp.float32),
               pltpu.VMEM((NBUF, C, D), jnp.float32)]
            + [pltpu.SemaphoreType.DMA for _ in range(2 * NBUF + 1)]
        ),
    )
    def run(idx_hbm, feat_hbm, self_hbm, mean_hbm, *scr):
        idx_vs = scr[:NBUF]
        rows_v, mean_v = scr[NBUF], scr[NBUF + 1]
        sems_g = scr[NBUF + 2:2 * NBUF + 2]
        sems_i = scr[2 * NBUF + 2:3 * NBUF + 2]
        sem_wb = scr[3 * NBUF + 2]
        wid = lax.axis_index("s") * NC + lax.axis_index("c")
        count = (Bp // C) // NW
        start = wid * count

        def fire_idx(ci, b):
            pltpu.async_copy(idx_hbm.at[start + ci], idx_vs[b], sems_i[b])

        def wait_idx(b):
            pltpu.make_async_copy(idx_hbm.at[start], idx_vs[b],
                                  sems_i[b]).wait()

        def fire_gather(b):
            pltpu.async_copy(feat_hbm.at[idx_vs[b]], rows_v.at[b], sems_g[b])

        def wait_gather(b):
            pltpu.make_async_copy(feat_hbm.at[idx_vs[b]], rows_v.at[b],
                                  sems_g[b]).wait()

        def fire_wb(ci, b):
            base = (start + ci) * C
            pltpu.async_copy(rows_v.at[b].at[pl.ds(0, C)],
                             self_hbm.at[pl.ds(base, C)], sem_wb)
            pltpu.async_copy(mean_v.at[b], mean_hbm.at[pl.ds(base, C)], sem_wb)

        def drain_wb(b):
            pltpu.make_async_copy(rows_v.at[b].at[pl.ds(0, C)],
                                  self_hbm.at[pl.ds(0, C)], sem_wb).wait()
            pltpu.make_async_copy(mean_v.at[b],
                                  mean_hbm.at[pl.ds(0, C)], sem_wb).wait()

        def compute(b):
            def row_body(r, rc):
                for g in range(D // 16):
                    s = pl.ds(g * 16, 16)
                    acc = rows_v[b, C + r, s]
                    for j in range(2, K + 1):
                        acc = acc + rows_v[b, j * C + r, s]
                    mean_v[b, r, s] = acc * jnp.float32(1.0 / K)
                return rc

            lax.fori_loop(0, C, row_body, 0)

        with jax.named_scope("sc_prolog"):
            pltpu.sync_copy(idx_hbm.at[start], idx_vs[0])
            for m in range(1, NBUF):
                fire_idx(m, m)
            fire_gather(0)
            for m in range(1, NBUF - 1):
                wait_idx(m)
                fire_gather(m)

        def group_body(i2, carry):
            for b in range(NBUF):
                ci = NBUF * i2 + b
                pb = (b - 1) % NBUF
                wait_gather(b)
                pl.when(ci + NBUF < count)(lambda: fire_idx(ci + NBUF, b))
                pl.when(ci >= 1)(lambda: drain_wb(pb))
                pl.when(ci + NBUF - 1 < count)(lambda: wait_idx(pb))
                pl.when(ci + NBUF - 1 < count)(lambda: fire_gather(pb))
                compute(b)
                fire_wb(ci, b)
            return carry

        with jax.named_scope("sc_loop"):
            lax.fori_loop(0, count // NBUF, group_body, 0)
        with jax.named_scope("sc_epilog"):
            drain_wb(NBUF - 1)

    return run(idx2, features)


def _mm_body(xs_ref, xm_ref, w1_ref, w2_ref, o_ref):
    dn = (((1,), (1,)), ((), ()))
    a = lax.dot_general(w1_ref[...], xs_ref[...], dn,
                        preferred_element_type=jnp.float32)
    b = lax.dot_general(w2_ref[...], xm_ref[...], dn,
                        preferred_element_type=jnp.float32)
    o_ref[...] = jnp.maximum(a + b, 0.0)


def _mm(xs, xm, w1, w2, B):
    nb = (B + BB - 1) // BB
    return pl.pallas_call(
        _mm_body,
        grid=(nb,),
        in_specs=[
            pl.BlockSpec((BB, D), lambda i: (i, 0)),
            pl.BlockSpec((BB, D), lambda i: (i, 0)),
            pl.BlockSpec((D, D), lambda i: (0, 0)),
            pl.BlockSpec((D, D), lambda i: (0, 0)),
        ],
        out_specs=pl.BlockSpec((D, BB), lambda i: (0, i)),
        out_shape=jax.ShapeDtypeStruct((D, B), jnp.float32),
    )(xs, xm, w1, w2)


def kernel(nodes, neigh_idx, features, W):
    B = nodes.shape[0]
    step = NW * NBUF * C
    Bp = ((B + step - 1) // step) * step
    nchunks_total = Bp // C

    idxT = jnp.concatenate(
        [nodes[None, :].astype(jnp.int32), neigh_idx.T.astype(jnp.int32)], axis=0)
    pad = (jnp.arange((K + 1) * (Bp - B), dtype=jnp.int32)
           % features.shape[0]).reshape(K + 1, Bp - B)
    idxT = jnp.concatenate([idxT, pad], axis=1)
    idx2 = (idxT.reshape(K + 1, nchunks_total, C)
            .transpose(1, 0, 2)
            .reshape(nchunks_total, (K + 1) * C))

    self_f, mean_f = _sc_gather_mean(idx2, features, Bp)
    return _mm(self_f, mean_f, W[:, :D], W[:, D:], B)


# speedup vs baseline: 1.1108x; 1.0955x over previous
import functools

import jax
import jax.numpy as jnp
from jax import lax
from jax.experimental import pallas as pl
from jax.experimental.pallas import tpu as pltpu
from jax.experimental.pallas import tpu_sc as plsc

D = 128
K = 10
NC = 2
NS = 16
NW = NC * NS
C = 24
BB = 8192
NBUF = 3


def _sc_gather_mean(idx1, features, Bp):
    mesh = plsc.VectorSubcoreMesh(core_axis_name="c", subcore_axis_name="s")

    @functools.partial(
        pl.kernel,
        out_type=(
            jax.ShapeDtypeStruct((Bp, D), jnp.float32),
            jax.ShapeDtypeStruct((Bp, D), jnp.float32),
        ),
        mesh=mesh,
        compiler_params=pltpu.CompilerParams(
            disable_bounds_checks=True, disable_semaphore_checks=True),
        scratch_types=(
            [pltpu.VMEM(((K + 1) * C,), jnp.int32) for _ in range(NBUF)]
            + [pltpu.VMEM((NBUF, (K + 1) * C, D), jnp.float32),
               pltpu.VMEM((NBUF, C, D), jnp.float32)]
            + [pltpu.SemaphoreType.DMA for _ in range(2 * NBUF + 1)]
        ),
    )
    def run(idx_hbm, feat_hbm, self_hbm, mean_hbm, *scr):
        idx_vs = scr[:NBUF]
        rows_v, mean_v = scr[NBUF], scr[NBUF + 1]
        sems_g = scr[NBUF + 2:2 * NBUF + 2]
        sems_i = scr[2 * NBUF + 2:3 * NBUF + 2]
        sem_wb = scr[3 * NBUF + 2]
        wid = lax.axis_index("s") * NC + lax.axis_index("c")
        count = (Bp // C) // NW
        start = wid * count

        def fire_idx(ci, b):
            base = (start + ci) * C
            for j in range(K + 1):
                pltpu.async_copy(idx_hbm.at[pl.ds(j * Bp + base, C)],
                                 idx_vs[b].at[pl.ds(j * C, C)], sems_i[b])

        def wait_idx(b):
            for j in range(K + 1):
                pltpu.make_async_copy(idx_hbm.at[pl.ds(j * Bp, C)],
                                      idx_vs[b].at[pl.ds(j * C, C)],
                                      sems_i[b]).wait()

        def fire_gather(b):
            pltpu.async_copy(feat_hbm.at[idx_vs[b]], rows_v.at[b], sems_g[b])

        def wait_gather(b):
            pltpu.make_async_copy(feat_hbm.at[idx_vs[b]], rows_v.at[b],
                                  sems_g[b]).wait()

        def fire_wb(ci, b):
            base = (start + ci) * C
            pltpu.async_copy(rows_v.at[b].at[pl.ds(0, C)],
                             self_hbm.at[pl.ds(base, C)], sem_wb)
            pltpu.async_copy(mean_v.at[b], mean_hbm.at[pl.ds(base, C)], sem_wb)

        def drain_wb(b):
            pltpu.make_async_copy(rows_v.at[b].at[pl.ds(0, C)],
                                  self_hbm.at[pl.ds(0, C)], sem_wb).wait()
            pltpu.make_async_copy(mean_v.at[b],
                                  mean_hbm.at[pl.ds(0, C)], sem_wb).wait()

        def compute(b):
            def row_body(r, rc):
                for g in range(D // 16):
                    s = pl.ds(g * 16, 16)
                    acc = rows_v[b, C + r, s]
                    for j in range(2, K + 1):
                        acc = acc + rows_v[b, j * C + r, s]
                    mean_v[b, r, s] = acc * jnp.float32(1.0 / K)
                return rc

            lax.fori_loop(0, C, row_body, 0)

        with jax.named_scope("sc_prolog"):
            for j in range(K + 1):
                pltpu.sync_copy(idx_hbm.at[pl.ds(j * Bp + start * C, C)],
                                idx_vs[0].at[pl.ds(j * C, C)])
            for m in range(1, NBUF):
                fire_idx(m, m)
            fire_gather(0)
            for m in range(1, NBUF - 1):
                wait_idx(m)
                fire_gather(m)

        def group_body(i2, carry):
            for b in range(NBUF):
                ci = NBUF * i2 + b
                pb = (b - 1) % NBUF
                wait_gather(b)
                pl.when(ci + NBUF < count)(lambda: fire_idx(ci + NBUF, b))
                pl.when(ci >= 1)(lambda: drain_wb(pb))
                pl.when(ci + NBUF - 1 < count)(lambda: wait_idx(pb))
                pl.when(ci + NBUF - 1 < count)(lambda: fire_gather(pb))
                compute(b)
                fire_wb(ci, b)
            return carry

        with jax.named_scope("sc_loop"):
            lax.fori_loop(0, count // NBUF, group_body, 0)
        with jax.named_scope("sc_epilog"):
            drain_wb(NBUF - 1)

    return run(idx1, features)


def _mm_body(xs_ref, xm_ref, w1_ref, w2_ref, o_ref):
    dn = (((1,), (1,)), ((), ()))
    a = lax.dot_general(w1_ref[...], xs_ref[...], dn,
                        preferred_element_type=jnp.float32)
    b = lax.dot_general(w2_ref[...], xm_ref[...], dn,
                        preferred_element_type=jnp.float32)
    o_ref[...] = jnp.maximum(a + b, 0.0)


def _mm(xs, xm, w1, w2, B):
    nb = (B + BB - 1) // BB
    return pl.pallas_call(
        _mm_body,
        grid=(nb,),
        in_specs=[
            pl.BlockSpec((BB, D), lambda i: (i, 0)),
            pl.BlockSpec((BB, D), lambda i: (i, 0)),
            pl.BlockSpec((D, D), lambda i: (0, 0)),
            pl.BlockSpec((D, D), lambda i: (0, 0)),
        ],
        out_specs=pl.BlockSpec((D, BB), lambda i: (0, i)),
        out_shape=jax.ShapeDtypeStruct((D, B), jnp.float32),
    )(xs, xm, w1, w2)


def kernel(nodes, neigh_idx, features, W):
    B = nodes.shape[0]
    step = NW * NBUF * C
    Bp = ((B + step - 1) // step) * step
    nchunks_total = Bp // C

    idxT = jnp.concatenate(
        [nodes[None, :].astype(jnp.int32), neigh_idx.T.astype(jnp.int32)], axis=0)
    pad = (jnp.arange((K + 1) * (Bp - B), dtype=jnp.int32)
           % features.shape[0]).reshape(K + 1, Bp - B)
    idxT = jnp.concatenate([idxT, pad], axis=1)
    idx1 = idxT.reshape((K + 1) * Bp)

    self_f, mean_f = _sc_gather_mean(idx1, features, Bp)
    return _mm(self_f, mean_f, W[:, :D], W[:, D:], B)
